# baseline (speedup 1.0000x reference)
import jax
import jax.numpy as jnp
from jax import lax
from jax.experimental import pallas as pl
from jax.experimental.pallas import tpu as pltpu
from jax.experimental.pallas import tpu_sc as plsc

B, C, W, H = 4, 96, 384, 384
NUM = 8
BW, BH = 64, 64
BC = B * C
NW = 32
TPW = BC // NW
CH = 96
NCHUNK = W // CH
NSTEP = TPW * NCHUNK
NBUF = 3
CHW = CH * H


def _fix_rects(buf, wshs_v, k, r0, zeros16, iota16):
    row16 = wshs_v[pl.ds(k * 16, 16)]
    for i in range(NUM):
        ws = jnp.minimum(jnp.maximum(row16[i], 0), W - 1)
        hs = jnp.minimum(jnp.maximum(row16[NUM + i], 0), H - 1)
        we = jnp.minimum(ws + BW, W - 1)
        he = jnp.minimum(hs + BH, H - 1)
        lo = jnp.maximum(ws - r0, 0)
        hi = jnp.maximum(jnp.minimum(we - r0, CH), lo)
        w0 = (hs // 16) * 16
        hsv = jnp.full((16,), hs, jnp.int32)
        hev = jnp.full((16,), he, jnp.int32)
        cols = []
        masks = []
        for kk in range(5):
            cv = w0 + kk * 16 + iota16
            m = (cv >= hsv) & (cv < hev)
            cols.append(jnp.minimum(cv, H - 1))
            masks.append(m)

        def row_body(r, acc):
            rb = r * H
            for kk in range(5):
                plsc.store_scatter(buf, [rb + cols[kk]], zeros16, mask=masks[kk])
            return acc

        lax.fori_loop(lo, hi, row_body, 0)


def _sc_kernel(x_hbm, wshs_hbm, out_hbm,
               b0, b1, b2, wshs_v,
               sin0, sin1, sin2, sout0, sout1, sout2):
    wid = lax.axis_index("s") * 2 + lax.axis_index("c")
    t0 = wid * TPW
    pltpu.sync_copy(wshs_hbm.at[pl.ds(t0 * 16, TPW * 16)], wshs_v)

    bufs = (b0, b1, b2)
    sins = (sin0, sin1, sin2)
    souts = (sout0, sout1, sout2)
    zeros16 = jnp.zeros((16,), jnp.float32)
    iota16 = lax.broadcasted_iota(jnp.int32, (16,), 0)

    def src_off(s):
        t = t0 + s // NCHUNK
        r0 = (s % NCHUNK) * CH
        return t * (W * H) + r0 * H

    def gather_start(s, u):
        pltpu.make_async_copy(x_hbm.at[pl.ds(src_off(s), CHW)], bufs[u], sins[u]).start()

    def gather_wait(u):
        pltpu.make_async_copy(x_hbm.at[pl.ds(0, CHW)], bufs[u], sins[u]).wait()

    def scatter_start(s, u):
        pltpu.make_async_copy(bufs[u], out_hbm.at[pl.ds(src_off(s), CHW)], souts[u]).start()

    def scatter_wait(u):
        pltpu.make_async_copy(bufs[u], out_hbm.at[pl.ds(0, CHW)], souts[u]).wait()

    gather_start(0, 0)
    gather_start(1, 1)

    def group_body(g, carry):
        for u in range(NBUF):
            s = g * NBUF + u
            gather_wait(u)
            k = s // NCHUNK
            r0 = (s % NCHUNK) * CH
            pass
            scatter_start(s, u)
            nxt = (u + 2) % NBUF

            if u == 0:
                @pl.when(s == 0)
                def _():
                    gather_start(2, 2)

            @pl.when((s >= 1) & (s + 2 < NSTEP))
            def _():
                scatter_wait(nxt)
                gather_start(s + 2, nxt)

        return carry

    lax.fori_loop(0, NSTEP // NBUF, group_body, 0)
    scatter_wait(0)
    scatter_wait(1)
    scatter_wait(2)


def kernel(x, width_start, height_start):
    x1 = x.reshape(BC * W * H)
    wshs = jnp.concatenate(
        [width_start.reshape(BC, NUM), height_start.reshape(BC, NUM)],
        axis=1).reshape(BC * 2 * NUM)
    mesh = plsc.VectorSubcoreMesh(core_axis_name="c", subcore_axis_name="s")
    run = pl.kernel(
        _sc_kernel,
        mesh=mesh,
        compiler_params=pltpu.CompilerParams(needs_layout_passes=False),
        out_type=jax.ShapeDtypeStruct((BC * W * H,), jnp.float32),
        scratch_types=[
            pltpu.VMEM((CHW,), jnp.float32),
            pltpu.VMEM((CHW,), jnp.float32),
            pltpu.VMEM((CHW,), jnp.float32),
            pltpu.VMEM((TPW * 2 * NUM,), jnp.int32),
            pltpu.SemaphoreType.DMA,
            pltpu.SemaphoreType.DMA,
            pltpu.SemaphoreType.DMA,
            pltpu.SemaphoreType.DMA,
            pltpu.SemaphoreType.DMA,
            pltpu.SemaphoreType.DMA,
        ],
    )
    out1 = run(x1, wshs)
    return out1.reshape(B, C, W, H)


# speedup vs baseline: 4.1541x; 4.1541x over previous
import jax
import jax.numpy as jnp
from jax import lax
from jax.experimental import pallas as pl
from jax.experimental.pallas import tpu as pltpu

B, C, W, H = 4, 96, 384, 384
NUM = 8
BW, BH = 64, 64
CH_BLK = 24


def _dropout_kernel(ws_ref, hs_ref, x_ref, o_ref):
    g = pl.program_id(0)
    b = g // (C // CH_BLK)
    c0 = (g % (C // CH_BLK)) * CH_BLK
    wi = lax.broadcasted_iota(jnp.int32, (W, NUM), 0)
    hi = lax.broadcasted_iota(jnp.int32, (NUM, H), 1)
    for ch in range(CH_BLK):
        c = c0 + ch
        ws = jnp.stack([jnp.clip(ws_ref[b, c, i], 0, W - 1) for i in range(NUM)])
        hs = jnp.stack([jnp.clip(hs_ref[b, c, i], 0, H - 1) for i in range(NUM)])
        we = jnp.minimum(ws + BW, W - 1)
        he = jnp.minimum(hs + BH, H - 1)
        R = ((wi >= ws[None, :]) & (wi < we[None, :])).astype(jnp.float32)
        Cm = ((hi >= hs[:, None]) & (hi < he[:, None])).astype(jnp.float32)
        M = jnp.dot(R, Cm, preferred_element_type=jnp.float32)
        o_ref[0, ch] = jnp.where(M > 0, jnp.float32(0), x_ref[0, ch])


def kernel(x, width_start, height_start):
    grid_spec = pltpu.PrefetchScalarGridSpec(
        num_scalar_prefetch=2,
        grid=(B * C // CH_BLK,),
        in_specs=[
            pl.BlockSpec(
                (1, CH_BLK, W, H),
                lambda i, ws, hs: (i // (C // CH_BLK), i % (C // CH_BLK), 0, 0),
            ),
        ],
        out_specs=pl.BlockSpec(
            (1, CH_BLK, W, H),
            lambda i, ws, hs: (i // (C // CH_BLK), i % (C // CH_BLK), 0, 0),
        ),
    )
    return pl.pallas_call(
        _dropout_kernel,
        grid_spec=grid_spec,
        out_shape=jax.ShapeDtypeStruct((B, C, W, H), jnp.float32),
        compiler_params=pltpu.CompilerParams(
            dimension_semantics=("parallel",),
        ),
    )(width_start, height_start, x)
